# baseline (speedup 1.0000x reference)
import functools

import jax
import jax.numpy as jnp
from jax.experimental import pallas as pl
from jax.experimental.pallas import tpu as pltpu


def _emd_block_kernel(p_ref, t_ref, o_ref, sp_acc, st_acc, tot_acc,
                      *, grid, inv_n, inv_b):
    i = pl.program_id(0)
    p = p_ref[:]
    t = t_ref[:]
    part_tot = jnp.sum(p * p + t * t, keepdims=True)
    part_sp = jnp.sum(p, axis=1, keepdims=True)
    part_st = jnp.sum(t, axis=1, keepdims=True)

    @pl.when(i == 0)
    def _init():
        tot_acc[:, :] = part_tot
        sp_acc[:, :] = part_sp
        st_acc[:, :] = part_st

    @pl.when(i > 0)
    def _accum():
        tot_acc[:, :] += part_tot
        sp_acc[:, :] += part_sp
        st_acc[:, :] += part_st

    @pl.when(i == grid - 1)
    def _finish():
        cross = jnp.sum(sp_acc[:, :] * st_acc[:, :], keepdims=True)
        o_ref[:, :] = (tot_acc[:, :] - 2.0 * inv_n * cross) * inv_b


def kernel(pred, target):
    b, n, c = pred.shape
    rows = b * c
    p = pred.transpose(0, 2, 1).reshape(rows, n)
    t = target.transpose(0, 2, 1).reshape(rows, n)
    grid = 4
    bn = n // grid
    out = pl.pallas_call(
        functools.partial(_emd_block_kernel, grid=grid,
                          inv_n=1.0 / n, inv_b=1.0 / b),
        grid=(grid,),
        in_specs=[
            pl.BlockSpec((rows, bn), lambda i: (0, i)),
            pl.BlockSpec((rows, bn), lambda i: (0, i)),
        ],
        out_specs=pl.BlockSpec((1, 1), lambda i: (0, 0)),
        out_shape=jax.ShapeDtypeStruct((1, 1), jnp.float32),
        scratch_shapes=[
            pltpu.VMEM((rows, 1), jnp.float32),
            pltpu.VMEM((rows, 1), jnp.float32),
            pltpu.VMEM((1, 1), jnp.float32),
        ],
    )(p, t)
    return out[0, 0]


# speedup vs baseline: 1.2589x; 1.2589x over previous
import functools

import jax
import jax.numpy as jnp
from jax.experimental import pallas as pl


def _emd_reduce_kernel(p_ref, t_ref, o_ref, *, inv_n, inv_b):
    p = p_ref[:]
    t = t_ref[:]
    q = jnp.sum(p * p + t * t, axis=1, keepdims=True)
    sp = jnp.sum(p, axis=1, keepdims=True)
    st = jnp.sum(t, axis=1, keepdims=True)
    comb = (q - (2.0 * inv_n) * sp * st) * inv_b
    o_ref[:, :] = jnp.sum(comb, axis=0, keepdims=True)


def kernel(pred, target):
    b, n, c = pred.shape
    p = pred.transpose(0, 2, 1).reshape(b * c, n)
    t = target.transpose(0, 2, 1).reshape(b * c, n)
    out = pl.pallas_call(
        functools.partial(_emd_reduce_kernel, inv_n=1.0 / n, inv_b=1.0 / b),
        out_shape=jax.ShapeDtypeStruct((1, 1), jnp.float32),
    )(p, t)
    return out[0, 0]
